# baseline (speedup 1.0000x reference)
import functools

import jax
import jax.numpy as jnp
from jax import lax
from jax.experimental import pallas as pl
from jax.experimental.pallas import tpu as pltpu
from jax.experimental.pallas import tpu_sc as plsc

_B = 4096
_D = 512
_K = 1000
_LANES = 16
_NC = 2
_NS = 16
_NW = _NC * _NS
_BPW = _B // _NW
_SCALE = 0.01 / _B

_mesh = plsc.VectorSubcoreMesh(core_axis_name="c", subcore_axis_name="s")


def _prep_tc_body(cent_ref, pn_ref, centb_ref):
    cf = cent_ref[...]
    g = (lax.broadcasted_iota(jnp.int32, (_D, _LANES), 0) % _LANES
         == lax.broadcasted_iota(jnp.int32, (_D, _LANES), 1)
         ).astype(jnp.float32)
    pn_ref[...] = lax.dot_general(
        cf * cf, g, (((1,), (0,)), ((), ())),
        preferred_element_type=jnp.float32)
    centb_ref[...] = cf.astype(jnp.bfloat16)


_prep_tc = pl.pallas_call(
    _prep_tc_body,
    out_shape=(
        jax.ShapeDtypeStruct((_K, _LANES), jnp.float32),
        jax.ShapeDtypeStruct((_K, _D), jnp.bfloat16),
    ),
)


@functools.partial(
    pl.kernel,
    out_type=jax.ShapeDtypeStruct((_NW, _LANES), jnp.float32),
    mesh=_mesh,
    scratch_types=[
        pltpu.VMEM((_BPW,), jnp.int32),
        pltpu.VMEM((_BPW, _LANES), jnp.float32),
        pltpu.VMEM((_LANES,), jnp.float32),
        pltpu.SemaphoreType.DMA,
    ],
    compiler_params=pltpu.CompilerParams(use_tc_tiling_on_sc=False),
)
def _lookup_sc(lab_hbm, pn_hbm, out_hbm, lab_v, rows_v, acc_v, sem):
    wid = lax.axis_index("s") * _NC + lax.axis_index("c")
    pltpu.sync_copy(lab_hbm.at[pl.ds(wid * _BPW, _BPW)], lab_v)
    pltpu.async_copy(pn_hbm.at[lab_v], rows_v, sem).wait()

    def body(i, tot):
        for u in range(8):
            tot = tot + rows_v[i * 8 + u, :]
        return tot

    tot = lax.fori_loop(0, _BPW // 8, body, jnp.zeros((_LANES,), jnp.float32))
    acc_v[...] = tot
    pltpu.sync_copy(acc_v, out_hbm.at[wid])


_BLK = 1024
_GRID = _B // _BLK


def _cross_tc_body(lab_ref, feat_ref, centb_ref, out_ref):
    i = pl.program_id(0)
    f = feat_ref[...]
    lab = lab_ref[0, 0, :]
    oh = (lab[:, None] == lax.broadcasted_iota(jnp.int32, (_BLK, _K), 1))
    cb = lax.dot_general(
        oh.astype(jnp.bfloat16), centb_ref[...],
        (((1,), (0,)), ((), ())), preferred_element_type=jnp.float32)
    s = jnp.sum(f * (f - 2.0 * cb))

    @pl.when(i == 0)
    def _init():
        out_ref[0, 0] = s

    @pl.when(i > 0)
    def _acc():
        out_ref[0, 0] = out_ref[0, 0] + s


_cross_tc = pl.pallas_call(
    _cross_tc_body,
    grid=(_GRID,),
    in_specs=[
        pl.BlockSpec((1, 1, _BLK), lambda i: (i, 0, 0)),
        pl.BlockSpec((_BLK, _D), lambda i: (i, 0)),
        pl.BlockSpec((_K, _D), lambda i: (0, 0)),
    ],
    out_specs=pl.BlockSpec((1, 1), lambda i: (0, 0),
                           memory_space=pltpu.SMEM),
    out_shape=jax.ShapeDtypeStruct((1, 1), jnp.float32),
)


def kernel(features, labels, centers):
    labels = labels.astype(jnp.int32)
    pn, centb = _prep_tc(centers)
    sc_part = _lookup_sc(labels, pn)
    tc_part = _cross_tc(labels.reshape(_GRID, 1, _BLK), features, centb)
    return _SCALE * (jnp.sum(sc_part) + tc_part[0, 0])
